# baseline (speedup 1.0000x reference)
import functools

import jax
import jax.numpy as jnp
from jax import lax
from jax.experimental import pallas as pl
from jax.experimental.pallas import tpu as pltpu
from jax.experimental.pallas import tpu_sc as plsc

MASKID = 0
NBUF = 4
LOOKAHEAD = 2


@functools.lru_cache(maxsize=None)
def _build(bsz, seq, vocab, dim):
    info = plsc.get_sparse_core_info()
    nw = info.num_cores * info.num_subcores
    rows_w = bsz // nw
    assert rows_w * nw == bsz
    assert rows_w % NBUF == 0 and LOOKAHEAD < NBUF

    mesh = plsc.VectorSubcoreMesh(core_axis_name="c", subcore_axis_name="s")

    @functools.partial(
        pl.kernel,
        mesh=mesh,
        compiler_params=pltpu.CompilerParams(use_tc_tiling_on_sc=True),
        out_type=jax.ShapeDtypeStruct((bsz, seq, dim), jnp.float32),
        scratch_types=[
            pltpu.VMEM((rows_w, seq), jnp.int32),
            [pltpu.VMEM((seq, dim), jnp.float32) for _ in range(NBUF)],
            [pltpu.SemaphoreType.DMA for _ in range(NBUF)],
            [pltpu.SemaphoreType.DMA for _ in range(NBUF)],
        ],
    )
    def emb(x_hbm, table_hbm, out_hbm, idx_v, bufs, isems, osems):
        wid = lax.axis_index("s") * info.num_cores + lax.axis_index("c")
        row0 = wid * rows_w

        pltpu.sync_copy(x_hbm.at[pl.ds(row0, rows_w)], idx_v)

        def gather(r, b):
            pltpu.async_copy(
                table_hbm.at[idx_v.at[r]], bufs[b], isems[b]
            )

        def drain_in(r, b):
            pltpu.make_async_copy(
                table_hbm.at[idx_v.at[r]], bufs[b], isems[b]
            ).wait()

        def copyout(r, b):
            pltpu.async_copy(bufs[b], out_hbm.at[row0 + r], osems[b])

        def drain_out(r, b):
            pltpu.make_async_copy(
                bufs[b], out_hbm.at[row0 + r], osems[b]
            ).wait()

        for r in range(LOOKAHEAD):
            gather(r, r % NBUF)

        for b in range(NBUF):
            r = b
            drain_in(r, b)
            copyout(r, b)
            rg = r + LOOKAHEAD
            bg = rg % NBUF
            if rg >= NBUF:
                drain_out(rg - NBUF, bg)
            gather(rg, bg)

        def body(g, _):
            for b in range(NBUF):
                r = g * NBUF + b
                drain_in(r, b)
                copyout(r, b)
                rg = r + LOOKAHEAD
                bg = (b + LOOKAHEAD) % NBUF

                @pl.when(rg < rows_w)
                def _():
                    drain_out(rg - NBUF, bg)
                    gather(rg, bg)

            return 0

        lax.fori_loop(1, rows_w // NBUF, body, 0)

        for b in range(NBUF):
            drain_out(rows_w - NBUF + b, b)

    return emb


def _mask_body(x_ref, mask_ref):
    mask_ref[...] = jnp.where(x_ref[...] != MASKID, 1, 0).astype(jnp.int32)


@functools.lru_cache(maxsize=None)
def _build_mask(bsz, seq):
    return pl.pallas_call(
        _mask_body,
        out_shape=jax.ShapeDtypeStruct((bsz, seq), jnp.int32),
    )


def kernel(x, table):
    bsz, seq = x.shape
    vocab, dim = table.shape
    xi = x.astype(jnp.int32)
    out = _build(bsz, seq, vocab, dim)(xi, table)
    mask = _build_mask(bsz, seq)(xi)
    return out, mask


# speedup vs baseline: 1.9412x; 1.9412x over previous
import functools

import jax
import jax.numpy as jnp
from jax import lax
from jax.experimental import pallas as pl
from jax.experimental.pallas import tpu as pltpu
from jax.experimental.pallas import tpu_sc as plsc

MASKID = 0
NBUF = 5
LOOKAHEAD = 3


@functools.lru_cache(maxsize=None)
def _build(bsz, seq, vocab, dim):
    info = plsc.get_sparse_core_info()
    nw = info.num_cores * info.num_subcores
    cols_w = bsz // nw
    assert cols_w * nw == bsz
    assert seq % NBUF == 0 and LOOKAHEAD < NBUF

    mesh = plsc.VectorSubcoreMesh(core_axis_name="c", subcore_axis_name="s")

    @functools.partial(
        pl.kernel,
        mesh=mesh,
        out_type=(
            jax.ShapeDtypeStruct((seq, bsz, dim), jnp.float32),
            jax.ShapeDtypeStruct((seq, bsz), jnp.int32),
        ),
        scratch_types=[
            pltpu.VMEM((seq, cols_w), jnp.int32),
            pltpu.VMEM((seq, cols_w), jnp.int32),
            [pltpu.VMEM((cols_w, dim), jnp.float32) for _ in range(NBUF)],
            [pltpu.SemaphoreType.DMA for _ in range(NBUF)],
            [pltpu.SemaphoreType.DMA for _ in range(NBUF)],
        ],
    )
    def emb(xt_hbm, table_hbm, out_hbm, mask_hbm,
            idx_v, mask_v, bufs, isems, osems):
        wid = lax.axis_index("s") * info.num_cores + lax.axis_index("c")
        col0 = wid * cols_w

        pltpu.sync_copy(xt_hbm.at[:, pl.ds(col0, cols_w)], idx_v)

        def gather(s, b):
            pltpu.async_copy(
                table_hbm.at[idx_v.at[s]], bufs[b], isems[b]
            )

        def drain_in(s, b):
            pltpu.make_async_copy(
                table_hbm.at[idx_v.at[s]], bufs[b], isems[b]
            ).wait()

        def copyout(s, b):
            pltpu.async_copy(
                bufs[b], out_hbm.at[s, pl.ds(col0, cols_w)], osems[b]
            )

        def drain_out(s, b):
            pltpu.make_async_copy(
                bufs[b], out_hbm.at[s, pl.ds(col0, cols_w)], osems[b]
            ).wait()

        for s in range(LOOKAHEAD):
            gather(s, s % NBUF)

        def mask_body(i, _):
            for j in range(cols_w // 16):
                v = idx_v[i, pl.ds(j * 16, 16)]
                mask_v[i, pl.ds(j * 16, 16)] = jnp.where(
                    v != MASKID, 1, 0
                ).astype(jnp.int32)
            return 0

        lax.fori_loop(0, seq, mask_body, 0)
        pltpu.sync_copy(mask_v, mask_hbm.at[:, pl.ds(col0, cols_w)])

        for b in range(NBUF):
            s = b
            drain_in(s, b)
            copyout(s, b)
            sg = s + LOOKAHEAD
            bg = sg % NBUF
            if sg >= NBUF:
                drain_out(sg - NBUF, bg)
            gather(sg, bg)

        def body(g, _):
            for b in range(NBUF):
                s = g * NBUF + b
                drain_in(s, b)
                copyout(s, b)
                sg = s + LOOKAHEAD
                bg = (b + LOOKAHEAD) % NBUF

                @pl.when(sg < seq)
                def _():
                    drain_out(sg - NBUF, bg)
                    gather(sg, bg)

            return 0

        lax.fori_loop(1, seq // NBUF, body, 0)

        for b in range(NBUF):
            drain_out(seq - NBUF + b, b)

    return emb


def kernel(x, table):
    bsz, seq = x.shape
    vocab, dim = table.shape
    xt = x.astype(jnp.int32).T
    out_t, mask_t = _build(bsz, seq, vocab, dim)(xt, table)
    return out_t.transpose(1, 0, 2), mask_t.T


# speedup vs baseline: 1.9448x; 1.0018x over previous
import functools

import jax
import jax.numpy as jnp
from jax import lax
from jax.experimental import pallas as pl
from jax.experimental.pallas import tpu as pltpu
from jax.experimental.pallas import tpu_sc as plsc

MASKID = 0
NBUF = 5
LOOKAHEAD = 3


@functools.lru_cache(maxsize=None)
def _build(bsz, seq, vocab, dim):
    info = plsc.get_sparse_core_info()
    nw = info.num_cores * info.num_subcores
    cols_w = bsz // nw
    assert cols_w * nw == bsz
    assert seq % NBUF == 0 and LOOKAHEAD < NBUF

    mesh = plsc.VectorSubcoreMesh(core_axis_name="c", subcore_axis_name="s")

    @functools.partial(
        pl.kernel,
        mesh=mesh,
        out_type=(
            jax.ShapeDtypeStruct((seq, bsz, dim), jnp.float32),
            jax.ShapeDtypeStruct((seq, bsz), jnp.int32),
        ),
        scratch_types=[
            pltpu.VMEM((seq, cols_w), jnp.int32),
            pltpu.VMEM((seq, cols_w), jnp.int32),
            [pltpu.VMEM((cols_w, dim), jnp.float32) for _ in range(NBUF)],
            [pltpu.SemaphoreType.DMA for _ in range(NBUF)],
            [pltpu.SemaphoreType.DMA for _ in range(NBUF)],
        ],
    )
    def emb(xt_hbm, table_hbm, out_hbm, mask_hbm,
            idx_v, mask_v, bufs, isems, osems):
        wid = lax.axis_index("s") * info.num_cores + lax.axis_index("c")
        col0 = wid * cols_w

        pltpu.sync_copy(xt_hbm.at[:, pl.ds(col0, cols_w)], idx_v)

        def gather(s, b):
            pltpu.async_copy(
                table_hbm.at[idx_v.at[s]], bufs[b], isems[b]
            )

        def drain_in(s, b):
            pltpu.make_async_copy(
                table_hbm.at[idx_v.at[s]], bufs[b], isems[b]
            ).wait()

        def copyout(s, b):
            pltpu.async_copy(
                bufs[b], out_hbm.at[s, pl.ds(col0, cols_w)], osems[b]
            )

        def drain_out(s, b):
            pltpu.make_async_copy(
                bufs[b], out_hbm.at[s, pl.ds(col0, cols_w)], osems[b]
            ).wait()

        for s in range(LOOKAHEAD):
            gather(s, s % NBUF)

        for b in range(NBUF):
            s = b
            drain_in(s, b)
            copyout(s, b)
            sg = s + LOOKAHEAD
            bg = sg % NBUF
            if sg >= NBUF:
                drain_out(sg - NBUF, bg)
            gather(sg, bg)

        def body(g, _):
            for b in range(NBUF):
                s = g * NBUF + b
                drain_in(s, b)
                copyout(s, b)
                sg = s + LOOKAHEAD
                bg = (b + LOOKAHEAD) % NBUF

                @pl.when(sg < seq)
                def _():
                    drain_out(sg - NBUF, bg)
                    gather(sg, bg)

            return 0

        lax.fori_loop(1, seq // NBUF, body, 0)

        def mask_body(i, _):
            for j in range(cols_w // 16):
                v = idx_v[i, pl.ds(j * 16, 16)]
                mask_v[i, pl.ds(j * 16, 16)] = jnp.where(
                    v != MASKID, 1, 0
                ).astype(jnp.int32)
            return 0

        lax.fori_loop(0, seq, mask_body, 0)
        pltpu.sync_copy(mask_v, mask_hbm.at[:, pl.ds(col0, cols_w)])

        for b in range(NBUF):
            drain_out(seq - NBUF + b, b)

    return emb


def kernel(x, table):
    bsz, seq = x.shape
    vocab, dim = table.shape
    xt = x.astype(jnp.int32).T
    out_t, mask_t = _build(bsz, seq, vocab, dim)(xt, table)
    return out_t.transpose(1, 0, 2), mask_t.T
